# baseline (speedup 1.0000x reference)
import functools

import jax
import jax.numpy as jnp
from jax import lax
from jax.experimental import pallas as pl
from jax.experimental.pallas import tpu as pltpu
from jax.experimental.pallas import tpu_sc as plsc

_NUM_TILES = 16
_ZROWS = 640
_CH = 128
_RING = 2
_GDEPTH = 1
_IDXB = 8


def _make_seg_sum(n, h, n_blocks):
    nacc = _NUM_TILES * _ZROWS
    assert nacc >= n + 1
    mesh = plsc.VectorSubcoreMesh(core_axis_name="c", subcore_axis_name="s",
                                  num_cores=2, num_subcores=_NUM_TILES)
    n_chunks = n_blocks * _IDXB
    assert n_chunks % _RING == 0
    assert (_IDXB // _RING) >= 2
    last = (_NUM_TILES - 1) * _ZROWS

    @functools.partial(
        pl.kernel,
        out_type=[jax.ShapeDtypeStruct((n, h), jnp.float32)] * 4,
        mesh=mesh,
        scratch_types=[
            pltpu.VMEM((2, _CH), jnp.int32),
            pltpu.VMEM((2, _CH), jnp.int32),
            pltpu.VMEM((_RING, _CH, h), jnp.float32),
            pltpu.VMEM_SHARED((nacc, h), jnp.float32),
            [pltpu.SemaphoreType.DMA] * _RING,
            pltpu.SemaphoreType.DMA,
        ],
    )
    def seg_sum(l_hbm, w_hbm, e_hbm, t_hbm, src_hbm, dst_hbm, z_hbm,
                aggl_hbm, aggw_hbm, agge_hbm, aggt_hbm,
                src_v, dst_v, rows_v, acc_sh, sem_g, sem_i):
        c = lax.axis_index("c")
        s = lax.axis_index("s")

        def src_row(cc):
            return src_v.at[(cc // _IDXB) % 2, cc % _IDXB]

        def dst_row(cc):
            return dst_v.at[(cc // _IDXB) % 2, cc % _IDXB]

        def process(feat_hbm, out_hbm):
            pltpu.sync_copy(z_hbm, acc_sh.at[pl.ds(s * _ZROWS, _ZROWS)])
            plsc.subcore_barrier()

            def group(g, carry):
                pltpu.sync_copy(src_hbm.at[s, pl.ds(g * 2, 2)], src_v)
                pltpu.sync_copy(dst_hbm.at[s, pl.ds(g * 2, 2)], dst_v)
                cp0 = pltpu.async_copy(feat_hbm.at[src_v.at[0]],
                                       rows_v.at[0], sem_g[0])
                cp1 = pltpu.async_copy(feat_hbm.at[src_v.at[1]],
                                       rows_v.at[1], sem_g[1])
                cp0.wait()
                sc0 = pltpu.async_copy(rows_v.at[0], acc_sh.at[dst_v.at[0]],
                                       sem_i, add=True)
                cp1.wait()
                pltpu.sync_copy(rows_v.at[1], acc_sh.at[dst_v.at[1]], add=True)
                sc0.wait()
                return carry

            lax.fori_loop(0, n_chunks // 2, group, 0, unroll=False)
            plsc.subcore_barrier()

            @pl.when(s < _NUM_TILES - 1)
            def _():
                sl = pl.ds(s * _ZROWS, _ZROWS)
                pltpu.sync_copy(acc_sh.at[sl], out_hbm.at[sl])

            @pl.when(s == _NUM_TILES - 1)
            def _():
                sl = pl.ds(last, n - last)
                pltpu.sync_copy(acc_sh.at[sl], out_hbm.at[sl])

            plsc.subcore_barrier()

        @pl.when(c == 0)
        def _():
            process(l_hbm, aggl_hbm)
            process(w_hbm, aggw_hbm)

        @pl.when(c == 1)
        def _():
            process(e_hbm, agge_hbm)
            process(t_hbm, aggt_hbm)

    return seg_sum


def _mlp_body(aggl_ref, aggw_ref, agge_ref, aggt_ref, w1_ref, b1_ref,
              wh_ref, bh_ref, g_ref, bt_ref, out_ref):
    h = aggl_ref.shape[1]
    x = jnp.dot(aggl_ref[...], w1_ref[0:h, :], preferred_element_type=jnp.float32)
    x = x + jnp.dot(aggw_ref[...], w1_ref[h:2 * h, :], preferred_element_type=jnp.float32)
    x = x + jnp.dot(agge_ref[...], w1_ref[2 * h:3 * h, :], preferred_element_type=jnp.float32)
    x = x + jnp.dot(aggt_ref[...], w1_ref[3 * h:4 * h, :], preferred_element_type=jnp.float32)
    x = jnp.maximum(x + b1_ref[...], 0.0)
    n = x.shape[0]
    mu = jnp.sum(x, axis=0, keepdims=True) / n
    xc = x - mu
    var = jnp.sum(xc * xc, axis=0, keepdims=True) / n
    y = xc * (g_ref[...] * lax.rsqrt(var + 1e-5)) + bt_ref[...]
    out_ref[...] = jnp.dot(y, wh_ref[...], preferred_element_type=jnp.float32) + bh_ref[...]


def kernel(l, w, e, t, edge_index, W1, b1, Wh, bh, gamma, beta):
    n, h = l.shape
    num_edges = edge_index.shape[1]

    blk_edges = _IDXB * _CH
    n_blocks = -(-num_edges // (_NUM_TILES * blk_edges))
    e_pad = n_blocks * blk_edges * _NUM_TILES
    pad = e_pad - num_edges
    src = jnp.concatenate([edge_index[0], jnp.zeros((pad,), jnp.int32)])
    dst = jnp.concatenate([edge_index[1], jnp.full((pad,), n, jnp.int32)])
    src3 = src.reshape(_NUM_TILES, n_blocks * _IDXB, _CH)
    dst3 = dst.reshape(_NUM_TILES, n_blocks * _IDXB, _CH)
    zeros = jnp.zeros((_ZROWS, h), jnp.float32)

    seg_sum = _make_seg_sum(n, h, n_blocks)
    aggl, aggw, agge, aggt = seg_sum(l, w, e, t, src3, dst3, zeros)

    l_new = pl.pallas_call(
        _mlp_body,
        out_shape=jax.ShapeDtypeStruct((n, h), jnp.float32),
    )(aggl, aggw, agge, aggt, W1, b1.reshape(1, h), Wh, bh.reshape(1, h),
      gamma.reshape(1, h), beta.reshape(1, h))

    return (l_new, aggw[:, None, :], agge[:, None, :], aggt[:, None, :])


# speedup vs baseline: 1.4597x; 1.4597x over previous
import functools

import jax
import jax.numpy as jnp
from jax import lax
from jax.experimental import pallas as pl
from jax.experimental.pallas import tpu as pltpu
from jax.experimental.pallas import tpu_sc as plsc

_NUM_TILES = 16
_ZROWS = 640
_CH = 128
_RING = 2
_GDEPTH = 1
_IDXB = 8


def _make_seg_sum(n, h, n_chunks):
    nacc = _NUM_TILES * _ZROWS
    assert nacc >= n + 1
    mesh = plsc.VectorSubcoreMesh(core_axis_name="c", subcore_axis_name="s",
                                  num_cores=2, num_subcores=_NUM_TILES)
    assert n_chunks % 2 == 0
    last = (_NUM_TILES - 1) * _ZROWS

    @functools.partial(
        pl.kernel,
        out_type=[jax.ShapeDtypeStruct((n, h), jnp.float32)] * 4,
        mesh=mesh,
        scratch_types=[
            pltpu.VMEM((4, _CH), jnp.int32),
            pltpu.VMEM((_RING, _CH, h), jnp.float32),
            pltpu.VMEM_SHARED((nacc, h), jnp.float32),
            [pltpu.SemaphoreType.DMA] * _RING,
        ],
    )
    def seg_sum(l_hbm, w_hbm, e_hbm, t_hbm, idx_hbm, z_hbm,
                aggl_hbm, aggw_hbm, agge_hbm, aggt_hbm,
                idx_v, rows_v, acc_sh, sem_g):
        c = lax.axis_index("c")
        s = lax.axis_index("s")

        def src_row(cc):
            return src_v.at[(cc // _IDXB) % 2, cc % _IDXB]

        def dst_row(cc):
            return dst_v.at[(cc // _IDXB) % 2, cc % _IDXB]

        def process(feat_hbm, out_hbm):
            pltpu.sync_copy(z_hbm, acc_sh.at[pl.ds(s * _ZROWS, _ZROWS)])
            plsc.subcore_barrier()

            def group(g, carry):
                pltpu.sync_copy(idx_hbm.at[s, g], idx_v)
                cp0 = pltpu.async_copy(feat_hbm.at[idx_v.at[0]],
                                       rows_v.at[0], sem_g[0])
                cp1 = pltpu.async_copy(feat_hbm.at[idx_v.at[1]],
                                       rows_v.at[1], sem_g[1])
                cp0.wait()
                pltpu.sync_copy(rows_v.at[0], acc_sh.at[idx_v.at[2]], add=True)
                cp1.wait()
                pltpu.sync_copy(rows_v.at[1], acc_sh.at[idx_v.at[3]], add=True)
                return carry

            lax.fori_loop(0, n_chunks // 2, group, 0, unroll=False)
            plsc.subcore_barrier()

            @pl.when(s < _NUM_TILES - 1)
            def _():
                sl = pl.ds(s * _ZROWS, _ZROWS)
                pltpu.sync_copy(acc_sh.at[sl], out_hbm.at[sl])

            @pl.when(s == _NUM_TILES - 1)
            def _():
                sl = pl.ds(last, n - last)
                pltpu.sync_copy(acc_sh.at[sl], out_hbm.at[sl])

            plsc.subcore_barrier()

        @pl.when(c == 0)
        def _():
            process(l_hbm, aggl_hbm)
            process(w_hbm, aggw_hbm)

        @pl.when(c == 1)
        def _():
            process(e_hbm, agge_hbm)
            process(t_hbm, aggt_hbm)

    return seg_sum


def _mlp_body(aggl_ref, aggw_ref, agge_ref, aggt_ref, w1_ref, b1_ref,
              wh_ref, bh_ref, g_ref, bt_ref, out_ref):
    h = aggl_ref.shape[1]
    x = jnp.dot(aggl_ref[...], w1_ref[0:h, :], preferred_element_type=jnp.float32)
    x = x + jnp.dot(aggw_ref[...], w1_ref[h:2 * h, :], preferred_element_type=jnp.float32)
    x = x + jnp.dot(agge_ref[...], w1_ref[2 * h:3 * h, :], preferred_element_type=jnp.float32)
    x = x + jnp.dot(aggt_ref[...], w1_ref[3 * h:4 * h, :], preferred_element_type=jnp.float32)
    x = jnp.maximum(x + b1_ref[...], 0.0)
    n = x.shape[0]
    mu = jnp.sum(x, axis=0, keepdims=True) / n
    xc = x - mu
    var = jnp.sum(xc * xc, axis=0, keepdims=True) / n
    y = xc * (g_ref[...] * lax.rsqrt(var + 1e-5)) + bt_ref[...]
    out_ref[...] = jnp.dot(y, wh_ref[...], preferred_element_type=jnp.float32) + bh_ref[...]


def kernel(l, w, e, t, edge_index, W1, b1, Wh, bh, gamma, beta):
    n, h = l.shape
    num_edges = edge_index.shape[1]

    grp_edges = 2 * _CH
    n_groups = -(-num_edges // (_NUM_TILES * grp_edges))
    e_pad = n_groups * grp_edges * _NUM_TILES
    pad = e_pad - num_edges
    src = jnp.concatenate([edge_index[0], jnp.zeros((pad,), jnp.int32)])
    dst = jnp.concatenate([edge_index[1], jnp.full((pad,), n, jnp.int32)])
    src4 = src.reshape(_NUM_TILES, n_groups, 2, _CH)
    dst4 = dst.reshape(_NUM_TILES, n_groups, 2, _CH)
    idx4 = jnp.concatenate([src4, dst4], axis=2)
    zeros = jnp.zeros((_ZROWS, h), jnp.float32)

    seg_sum = _make_seg_sum(n, h, 2 * n_groups)
    aggl, aggw, agge, aggt = seg_sum(l, w, e, t, idx4, zeros)

    l_new = pl.pallas_call(
        _mlp_body,
        out_shape=jax.ShapeDtypeStruct((n, h), jnp.float32),
    )(aggl, aggw, agge, aggt, W1, b1.reshape(1, h), Wh, bh.reshape(1, h),
      gamma.reshape(1, h), beta.reshape(1, h))

    return (l_new, aggw[:, None, :], agge[:, None, :], aggt[:, None, :])
